# Initial kernel scaffold; baseline (speedup 1.0000x reference)
import jax
import jax.numpy as jnp
from jax import lax
from jax.experimental import pallas as pl
from jax.experimental.pallas import tpu as pltpu
from jax.experimental.pallas import tpu_sc as plsc

DIM = 64
VOCAB = 1000000
BATCH = 4096
HIST = 50

L = 16
NW = 32
TOTAL = BATCH * HIST
IDX_COLS = 128
IDX_ROWS = TOTAL // IDX_COLS
ROWS_PER_W = IDX_ROWS // NW


def _body(idx_hbm, kern_hbm, priori_hbm, out_hbm,
          idx_v, idxp_v, rows_v, kern_v, gsem):
    wid = lax.axis_index("s") * 2 + lax.axis_index("c")
    pltpu.sync_copy(kern_hbm, kern_v)

    def step(m, carry):
        r = wid * ROWS_PER_W + m
        pltpu.sync_copy(idx_hbm.at[r], idx_v)

        def adj(k, c):
            v = idx_v[pl.ds(k * L, L)]
            idxp_v[pl.ds(k * L, L)] = jnp.maximum(v - 2, 0)
            return c
        lax.fori_loop(0, IDX_COLS // L, adj, 0)

        pltpu.async_copy(priori_hbm.at[idxp_v], rows_v, gsem).wait()

        def patch(k, c):
            v = idx_v[pl.ds(k * L, L)]
            mn = jnp.min(v)

            @pl.when(mn < 2)
            def _patch():
                msk = v < 2
                cidx = jnp.minimum(v, 1)
                row_ids = k * L + lax.iota(jnp.int32, L)

                def col(d, cc):
                    dv = jnp.full((L,), d, jnp.int32)
                    vals = plsc.load_gather(kern_v, (cidx, dv))
                    plsc.store_scatter(rows_v, (row_ids, dv), vals, mask=msk)
                    return cc
                lax.fori_loop(0, DIM, col, 0)
            return c
        lax.fori_loop(0, IDX_COLS // L, patch, 0)

        pltpu.sync_copy(rows_v, out_hbm.at[pl.ds(r * IDX_COLS, IDX_COLS)])
        return carry
    lax.fori_loop(0, ROWS_PER_W, step, 0)


@jax.jit
def kernel(inputs, kernel, priori):
    idx = inputs.reshape(-1).astype(jnp.int32).reshape(IDX_ROWS, IDX_COLS)
    mesh = plsc.VectorSubcoreMesh(core_axis_name="c", subcore_axis_name="s")
    k = pl.kernel(
        _body,
        out_type=jax.ShapeDtypeStruct((TOTAL, DIM), jnp.float32),
        mesh=mesh,
        scratch_types=[
            pltpu.VMEM((IDX_COLS,), jnp.int32),
            pltpu.VMEM((IDX_COLS,), jnp.int32),
            pltpu.VMEM((IDX_COLS, DIM), jnp.float32),
            pltpu.VMEM((2, DIM), jnp.float32),
            pltpu.SemaphoreType.DMA,
        ],
    )
    out = k(idx, kernel, priori)
    return out.reshape(BATCH, HIST, DIM)


# speedup vs baseline: 1.2553x; 1.2553x over previous
import jax
import jax.numpy as jnp
from jax import lax
from jax.experimental import pallas as pl
from jax.experimental.pallas import tpu as pltpu
from jax.experimental.pallas import tpu_sc as plsc

DIM = 64
VOCAB = 1000000
BATCH = 4096
HIST = 50

L = 16
NW = 32
TOTAL = BATCH * HIST
IDX_COLS = 128
IDX_ROWS = TOTAL // IDX_COLS
ROWS_PER_W = IDX_ROWS // NW


def _body(idx_hbm, kern_hbm, priori_hbm, out_hbm,
          idx_v, idxp_v, rows_v, kern_v, gsem):
    wid = lax.axis_index("s") * 2 + lax.axis_index("c")
    pltpu.sync_copy(kern_hbm, kern_v)

    def step(m, carry):
        r = wid * ROWS_PER_W + m
        pltpu.sync_copy(idx_hbm.at[r], idx_v)

        def adj(k, c):
            v = idx_v[pl.ds(k * L, L)]
            idxp_v[pl.ds(k * L, L)] = jnp.maximum(v - 2, 0)
            return c
        lax.fori_loop(0, IDX_COLS // L, adj, 0)

        pltpu.async_copy(priori_hbm.at[idxp_v], rows_v, gsem).wait()

        def patch(k, c):
            v = idx_v[pl.ds(k * L, L)]

            @pl.when(jnp.any(v < 2))
            def _patch():
                msk = v < 2
                cidx = jnp.minimum(v, 1)
                row_ids = k * L + lax.iota(jnp.int32, L)

                def col(d, cc):
                    dv = jnp.full((L,), d, jnp.int32)
                    vals = plsc.load_gather(kern_v, (cidx, dv))
                    plsc.store_scatter(rows_v, (row_ids, dv), vals, mask=msk)
                    return cc
                lax.fori_loop(0, DIM, col, 0)
            return c
        lax.fori_loop(0, IDX_COLS // L, patch, 0)

        pltpu.sync_copy(rows_v, out_hbm.at[pl.ds(r * IDX_COLS, IDX_COLS)])
        return carry
    lax.fori_loop(0, ROWS_PER_W, step, 0)


@jax.jit
def kernel(inputs, kernel, priori):
    idx = inputs.reshape(-1).astype(jnp.int32).reshape(IDX_ROWS, IDX_COLS)
    mesh = plsc.VectorSubcoreMesh(core_axis_name="c", subcore_axis_name="s")
    k = pl.kernel(
        _body,
        out_type=jax.ShapeDtypeStruct((TOTAL, DIM), jnp.float32),
        mesh=mesh,
        compiler_params=pltpu.CompilerParams(
            needs_layout_passes=False, use_tc_tiling_on_sc=False),
        scratch_types=[
            pltpu.VMEM((IDX_COLS,), jnp.int32),
            pltpu.VMEM((IDX_COLS,), jnp.int32),
            pltpu.VMEM((IDX_COLS, DIM), jnp.float32),
            pltpu.VMEM((2, DIM), jnp.float32),
            pltpu.SemaphoreType.DMA,
        ],
    )
    out = k(idx, kernel, priori)
    return out.reshape(BATCH, HIST, DIM)
